# Initial kernel scaffold; baseline (speedup 1.0000x reference)
import jax
import jax.numpy as jnp
from jax.experimental import pallas as pl


def kernel(h):
    raise NotImplementedError("write your pallas kernel here")


# speedup vs baseline: 10.2660x; 10.2660x over previous
import functools

import jax
import jax.numpy as jnp
from jax import lax
from jax.experimental import pallas as pl

N = 4096
D = 128
K = 10
THETA = 5.0
BLK = 256
NEG = -3.0e38


def _fused_kernel(h_ref, hb_ref, o_ref):
    h = h_ref[...]
    norm = jnp.sqrt(jnp.sum(h * h, axis=1, keepdims=True))
    hn = h / jnp.maximum(norm, 1e-8)

    hb = hb_ref[...]
    nb = jnp.sqrt(jnp.sum(hb * hb, axis=1, keepdims=True))
    hn_blk = hb / jnp.maximum(nb, 1e-8)

    scores = jnp.dot(hn_blk, hn.T, preferred_element_type=jnp.float32)

    col = lax.broadcasted_iota(jnp.int32, (BLK, N), 1)
    s = scores
    w = jnp.zeros((BLK, N), jnp.float32)
    v0 = None
    denom = jnp.zeros((BLK, 1), jnp.float32)
    for k in range(K):
        m = jnp.max(s, axis=1, keepdims=True)
        if k == 0:
            v0 = m
        idx = jnp.min(jnp.where(s == m, col, N), axis=1, keepdims=True)
        sel = col == idx
        wk = jnp.exp(THETA * (m - v0))
        w = jnp.where(sel, wk, w)
        denom = denom + wk
        if k < K - 1:
            s = jnp.where(sel, NEG, s)

    w = w / denom
    o_ref[...] = jnp.dot(w, h, preferred_element_type=jnp.float32)


def kernel(h):
    return pl.pallas_call(
        _fused_kernel,
        grid=(N // BLK,),
        in_specs=[
            pl.BlockSpec((N, D), lambda i: (0, 0)),
            pl.BlockSpec((BLK, D), lambda i: (i, 0)),
        ],
        out_specs=pl.BlockSpec((BLK, D), lambda i: (i, 0)),
        out_shape=jax.ShapeDtypeStruct((N, D), jnp.float32),
    )(h, h)


# speedup vs baseline: 21.0326x; 2.0488x over previous
import jax
import jax.numpy as jnp
from jax.experimental import pallas as pl
from jax.experimental.pallas import tpu as pltpu

N = 4096
D = 128
K = 10
THETA = 5.0
BLK = 256
NEG = -3.0e38


def _fused_kernel(h_ref, hb_ref, o_ref):
    h = h_ref[...]
    norm = jnp.sqrt(jnp.sum(h * h, axis=1, keepdims=True))
    hn = h / jnp.maximum(norm, 1e-8)

    hb = hb_ref[...]
    nb = jnp.sqrt(jnp.sum(hb * hb, axis=1, keepdims=True))
    hn_blk = hb / jnp.maximum(nb, 1e-8)

    s = jnp.dot(hn_blk, hn.T, preferred_element_type=jnp.float32)

    m = jnp.max(s, axis=1, keepdims=True)
    v0 = m
    for _ in range(K - 1):
        m = jnp.max(jnp.where(s < m, s, NEG), axis=1, keepdims=True)
    t = m

    e = jnp.where(s >= t, jnp.exp(THETA * (s - v0)), 0.0)
    denom = jnp.sum(e, axis=1, keepdims=True)
    o_ref[...] = jnp.dot(e, h, preferred_element_type=jnp.float32) / denom


def kernel(h):
    return pl.pallas_call(
        _fused_kernel,
        grid=(N // BLK,),
        in_specs=[
            pl.BlockSpec((N, D), lambda i: (0, 0)),
            pl.BlockSpec((BLK, D), lambda i: (i, 0)),
        ],
        out_specs=pl.BlockSpec((BLK, D), lambda i: (i, 0)),
        out_shape=jax.ShapeDtypeStruct((N, D), jnp.float32),
        compiler_params=pltpu.CompilerParams(
            dimension_semantics=("parallel",),
        ),
    )(h, h)


# speedup vs baseline: 25.6055x; 1.2174x over previous
import jax
import jax.numpy as jnp
from jax.experimental import pallas as pl
from jax.experimental.pallas import tpu as pltpu

N = 4096
D = 128
K = 10
THETA = 5.0
BLK = 256
NEG = -3.0e38


LEAF_W = 512


def _distinct_maxima(s, k):
    out = []
    m = jnp.max(s, axis=1, keepdims=True)
    out.append(m)
    for _ in range(k - 1):
        m = jnp.max(jnp.where(s < m, s, NEG), axis=1, keepdims=True)
        out.append(m)
    return out


def _topk_candidates(s, k):
    if s.shape[1] > LEAF_W and k >= 2:
        half = s.shape[1] // 2
        a = s[:, :half]
        b = s[:, half:]
        hi = jnp.maximum(a, b)
        lo = jnp.minimum(a, b)
        return _topk_candidates(hi, k) + _topk_candidates(lo, (k + 1) // 2)
    return _distinct_maxima(s, k)


def _fused_kernel(h_ref, hb_ref, o_ref):
    h = h_ref[...]
    norm = jnp.sqrt(jnp.sum(h * h, axis=1, keepdims=True))
    hn = h / jnp.maximum(norm, 1e-8)

    hb = hb_ref[...]
    nb = jnp.sqrt(jnp.sum(hb * hb, axis=1, keepdims=True))
    hn_blk = hb / jnp.maximum(nb, 1e-8)

    s = jnp.dot(hn_blk, hn.T, preferred_element_type=jnp.float32)

    cand = jnp.concatenate(_topk_candidates(s, K), axis=1)
    t = _distinct_maxima(cand, K)[K - 1]

    e = jnp.where(s >= t, jnp.exp(THETA * s), 0.0)
    denom = jnp.sum(e, axis=1, keepdims=True)
    o_ref[...] = jnp.dot(e, h, preferred_element_type=jnp.float32) / denom


def kernel(h):
    return pl.pallas_call(
        _fused_kernel,
        grid=(N // BLK,),
        in_specs=[
            pl.BlockSpec((N, D), lambda i: (0, 0)),
            pl.BlockSpec((BLK, D), lambda i: (i, 0)),
        ],
        out_specs=pl.BlockSpec((BLK, D), lambda i: (i, 0)),
        out_shape=jax.ShapeDtypeStruct((N, D), jnp.float32),
        compiler_params=pltpu.CompilerParams(
            dimension_semantics=("parallel",),
        ),
    )(h, h)
